# baseline (speedup 1.0000x reference)
import functools

import jax
import jax.numpy as jnp
from jax import lax
from jax.experimental import pallas as pl
from jax.experimental.pallas import tpu as pltpu
from jax.experimental.pallas import tpu_sc as plsc

B, N, D = 32, 576, 384
NC, NS, LANES = 2, 16, 16
NW = NC * NS
ROWS = N // NW
SLICE = ROWS * D
BATCH_STRIDE = N * D

_mesh = plsc.VectorSubcoreMesh(core_axis_name="c", subcore_axis_name="s")


@functools.partial(
    pl.kernel,
    mesh=_mesh,
    out_type=jax.ShapeDtypeStruct((B * N * D,), jnp.float32),
    scratch_types=[
        pltpu.VMEM((SLICE,), jnp.float32),
        pltpu.VMEM((SLICE,), jnp.float32),
        pltpu.SemaphoreType.DMA,
    ],
)
def _sc_add(x_hbm, p_hbm, o_hbm, pos_v, buf, sem):
    wid = lax.axis_index("s") * NC + lax.axis_index("c")
    base = wid * SLICE
    pltpu.sync_copy(p_hbm.at[pl.ds(base, SLICE)], pos_v)

    def batch_body(b, _):
        off = b * BATCH_STRIDE + base
        pltpu.sync_copy(x_hbm.at[pl.ds(off, SLICE)], buf)

        def add_body(i, _):
            s = pl.ds(i * LANES, LANES)
            buf[s] = buf[s] + pos_v[s]
            return 0

        lax.fori_loop(0, SLICE // LANES, add_body, 0)
        pltpu.sync_copy(buf, o_hbm.at[pl.ds(off, SLICE)])
        return 0

    lax.fori_loop(0, B, batch_body, 0)


def kernel(inputs, pos_table):
    x = inputs.reshape(B * N * D)
    p = pos_table.reshape(N * D)
    out = _sc_add(x, p)
    return out.reshape(B, N, D)


# speedup vs baseline: 1.7105x; 1.7105x over previous
import functools

import jax
import jax.numpy as jnp
from jax import lax
from jax.experimental import pallas as pl
from jax.experimental.pallas import tpu as pltpu
from jax.experimental.pallas import tpu_sc as plsc

B, N, D = 32, 576, 384
NC, NS, LANES = 2, 16, 16
NW = NC * NS
ROWS = N // NW
SLICE = ROWS * D
BATCH_STRIDE = N * D
NB = 4

_mesh = plsc.VectorSubcoreMesh(core_axis_name="c", subcore_axis_name="s")


@functools.partial(
    pl.kernel,
    mesh=_mesh,
    out_type=jax.ShapeDtypeStruct((B * N * D,), jnp.float32),
    scratch_types=(
        [pltpu.VMEM((SLICE,), jnp.float32)] * (1 + NB)
        + [pltpu.SemaphoreType.DMA] * (2 * NB)
    ),
)
def _sc_add(x_hbm, p_hbm, o_hbm, pos_v, *rest):
    bufs = rest[:NB]
    gsems = rest[NB:2 * NB]
    ssems = rest[2 * NB:]

    wid = lax.axis_index("s") * NC + lax.axis_index("c")
    base = wid * SLICE
    pltpu.sync_copy(p_hbm.at[pl.ds(base, SLICE)], pos_v)

    def start_gather(b, j):
        pltpu.make_async_copy(
            x_hbm.at[pl.ds(b * BATCH_STRIDE + base, SLICE)], bufs[j], gsems[j]
        ).start()

    def start_scatter(b, j):
        pltpu.make_async_copy(
            bufs[j], o_hbm.at[pl.ds(b * BATCH_STRIDE + base, SLICE)], ssems[j]
        ).start()

    def wait_gather(j):
        pltpu.make_async_copy(
            x_hbm.at[pl.ds(0, SLICE)], bufs[j], gsems[j]
        ).wait()

    def wait_scatter(j):
        pltpu.make_async_copy(
            bufs[j], o_hbm.at[pl.ds(0, SLICE)], ssems[j]
        ).wait()

    start_gather(0, 0)

    def round_body(k, _):
        for j in range(NB):
            b = k * NB + j
            jn = (j + 1) % NB

            @pl.when(b + 1 < B)
            def _prefetch():
                @pl.when(b + 1 >= NB)
                def _drain():
                    wait_scatter(jn)

                start_gather(b + 1, jn)

            wait_gather(j)

            @plsc.parallel_loop(0, SLICE, step=LANES, unroll=8)
            def _add(i):
                s = pl.ds(i, LANES)
                plsc.addupdate(bufs[j].at[s], pos_v[s])

            start_scatter(b, j)
        return 0

    lax.fori_loop(0, B // NB, round_body, 0)
    for j in range(NB):
        wait_scatter(j)


def kernel(inputs, pos_table):
    x = inputs.reshape(B * N * D)
    p = pos_table.reshape(N * D)
    out = _sc_add(x, p)
    return out.reshape(B, N, D)


# speedup vs baseline: 5.8100x; 3.3968x over previous
import jax
import jax.numpy as jnp
from jax.experimental import pallas as pl
from jax.experimental.pallas import tpu as pltpu

B, N, D = 32, 576, 384


def _body(x_ref, p_ref, o_ref):
    o_ref[...] = x_ref[...] + p_ref[...]


def kernel(inputs, pos_table):
    return pl.pallas_call(
        _body,
        grid=(B,),
        in_specs=[
            pl.BlockSpec((1, N, D), lambda b: (b, 0, 0)),
            pl.BlockSpec((N, D), lambda b: (0, 0)),
        ],
        out_specs=pl.BlockSpec((1, N, D), lambda b: (b, 0, 0)),
        out_shape=jax.ShapeDtypeStruct((B, N, D), jnp.float32),
    )(inputs, pos_table)
